# Initial kernel scaffold; baseline (speedup 1.0000x reference)
import jax
import jax.numpy as jnp
from jax.experimental import pallas as pl


def kernel(x, W1_l, W1_r, b1, g1, be1, W2_l, W2_r, b2, g2, be2, edge_index):
    raise NotImplementedError("write your pallas kernel here")


# speedup vs baseline: 4.2646x; 4.2646x over previous
import functools

import jax
import jax.numpy as jnp
from jax import lax
from jax.experimental import pallas as pl
from jax.experimental.pallas import tpu as pltpu
from jax.experimental.pallas import tpu_sc as plsc

N_NODES = 10000
D = 128

NC = 2
NS = 16
NW = NC * NS
CHUNK = 128

N_ACC = 10112
ROW_BLK = 1000


def _make_sc_segsum(ne_pad, with_deg):
    epw = ne_pad // NW
    n_chunks = epw // CHUNK
    slab = N_ACC // NS
    last_rows = N_NODES - (NS - 1) * slab
    dslab = 640
    dlast = N_ACC - (NS - 1) * dslab
    mesh = plsc.VectorSubcoreMesh(core_axis_name="c", subcore_axis_name="s")

    out_type = [jax.ShapeDtypeStruct((NC, N_NODES, D), jnp.float32)]
    scratch = [
        pltpu.VMEM((CHUNK,), jnp.int32),
        pltpu.VMEM((CHUNK,), jnp.int32),
        pltpu.VMEM((CHUNK, D), jnp.float32),
        pltpu.VMEM_SHARED((N_ACC, D), jnp.float32),
        pltpu.SemaphoreType.DMA,
    ]
    if with_deg:
        out_type.append(jax.ShapeDtypeStruct((NC, N_ACC), jnp.float32))
        scratch += [
            pltpu.VMEM((CHUNK,), jnp.float32),
            pltpu.VMEM_SHARED((N_ACC,), jnp.float32),
        ]

    @functools.partial(pl.kernel, out_type=out_type, mesh=mesh,
                       scratch_types=scratch)
    def segsum(table_hbm, src_hbm, dst_hbm, zeros_hbm, ones_hbm, zeros1_hbm,
               out_hbm, *rest):
        if with_deg:
            deg_hbm, src_v, dst_v, rows_v, acc_sh, sem, ones_v, dacc_sh = rest
        else:
            src_v, dst_v, rows_v, acc_sh, sem = rest
        c = lax.axis_index("c")
        s = lax.axis_index("s")
        zb = s * slab
        pltpu.sync_copy(zeros_hbm.at[pl.ds(zb, slab)],
                        acc_sh.at[pl.ds(zb, slab)])
        if with_deg:
            db = s * dslab

            @pl.when(s < NS - 1)
            def _():
                pltpu.sync_copy(zeros1_hbm.at[pl.ds(db, dslab)],
                                dacc_sh.at[pl.ds(db, dslab)])

            @pl.when(s == NS - 1)
            def _():
                dlb = (NS - 1) * dslab
                pltpu.sync_copy(zeros1_hbm.at[pl.ds(dlb, dlast)],
                                dacc_sh.at[pl.ds(dlb, dlast)])

            pltpu.sync_copy(ones_hbm, ones_v)
        plsc.subcore_barrier()
        base0 = (c * NS + s) * epw

        @pl.loop(0, n_chunks)
        def _(i):
            base = base0 + i * CHUNK
            pltpu.sync_copy(src_hbm.at[pl.ds(base, CHUNK)], src_v)
            pltpu.sync_copy(dst_hbm.at[pl.ds(base, CHUNK)], dst_v)
            pltpu.async_copy(table_hbm.at[src_v], rows_v, sem).wait()
            pltpu.sync_copy(rows_v, acc_sh.at[dst_v], add=True)
            if with_deg:
                pltpu.sync_copy(ones_v, dacc_sh.at[dst_v], add=True)

        plsc.subcore_barrier()
        ob = s * slab

        @pl.when(s < NS - 1)
        def _():
            pltpu.sync_copy(acc_sh.at[pl.ds(ob, slab)],
                            out_hbm.at[c].at[pl.ds(ob, slab)])

        @pl.when(s == NS - 1)
        def _():
            lb = (NS - 1) * slab
            pltpu.sync_copy(acc_sh.at[pl.ds(lb, last_rows)],
                            out_hbm.at[c].at[pl.ds(lb, last_rows)])

        if with_deg:
            db = s * dslab

            @pl.when(s < NS - 1)
            def _():
                pltpu.sync_copy(dacc_sh.at[pl.ds(db, dslab)],
                                deg_hbm.at[c].at[pl.ds(db, dslab)])

            @pl.when(s == NS - 1)
            def _():
                dlb = (NS - 1) * dslab
                pltpu.sync_copy(dacc_sh.at[pl.ds(dlb, dlast)],
                                deg_hbm.at[c].at[pl.ds(dlb, dlast)])

    return segsum


def _tc1_body(x_ref, wl_ref, wr_ref, b_ref, table_ref, xr_ref):
    x = x_ref[...]
    table_ref[...] = jnp.dot(x, wl_ref[...],
                             preferred_element_type=jnp.float32,
                             precision=lax.Precision.HIGHEST)
    xr_ref[...] = jnp.dot(x, wr_ref[...], preferred_element_type=jnp.float32,
                          precision=lax.Precision.HIGHEST) + b_ref[...]


def _tc2_body(p0_ref, p1_ref, d0_ref, d1_ref, xr_ref, g1_ref, be1_ref,
              w2l_ref, w2r_ref, b2_ref, hl_ref, hr_ref):
    agg = p0_ref[...] + p1_ref[...]
    deg = d0_ref[...] + d1_ref[...]
    scale = 1.0 / jnp.maximum(deg, 1.0)
    pre = agg * scale + xr_ref[...]
    mu = jnp.mean(pre, axis=-1, keepdims=True)
    var = jnp.mean((pre - mu) ** 2, axis=-1, keepdims=True)
    ln = (pre - mu) / jnp.sqrt(var + 1e-5) * g1_ref[...] + be1_ref[...]
    h = jax.nn.gelu(ln)
    hl_ref[...] = jnp.dot(h, w2l_ref[...], preferred_element_type=jnp.float32,
                          precision=lax.Precision.HIGHEST)
    hr_ref[...] = jnp.dot(h, w2r_ref[...], preferred_element_type=jnp.float32,
                          precision=lax.Precision.HIGHEST) + b2_ref[...]


def _tc3_body(q0_ref, q1_ref, d0_ref, d1_ref, hr_ref, g2_ref, be2_ref,
              out_ref):
    deg = d0_ref[...] + d1_ref[...]
    scale = 1.0 / jnp.maximum(deg, 1.0)
    pre = (q0_ref[...] + q1_ref[...]) * scale + hr_ref[...]
    mu = jnp.mean(pre, axis=-1, keepdims=True)
    var = jnp.mean((pre - mu) ** 2, axis=-1, keepdims=True)
    out_ref[...] = (pre - mu) / jnp.sqrt(var + 1e-5) * g2_ref[...] + be2_ref[...]


def _row_spec(width):
    return pl.BlockSpec((ROW_BLK, width), lambda i: (i, 0))


def _full_spec(shape):
    return pl.BlockSpec(shape, lambda i: (0, 0))


def kernel(x, W1_l, W1_r, b1, g1, be1, W2_l, W2_r, b2, g2, be2, edge_index):
    n = x.shape[0]
    ne = edge_index.shape[1]
    grid = (n // ROW_BLK,)

    src = edge_index[0].astype(jnp.int32)
    dst = edge_index[1].astype(jnp.int32)
    ne_pad = ((ne + NW * CHUNK - 1) // (NW * CHUNK)) * (NW * CHUNK)
    pad = ne_pad - ne
    src_p = jnp.concatenate([src, jnp.zeros((pad,), jnp.int32)])
    dst_p = jnp.concatenate([dst, jnp.full((pad,), n, jnp.int32)])

    zeros2d = jnp.zeros((N_ACC, D), jnp.float32)
    zeros1d = jnp.zeros((N_ACC,), jnp.float32)
    ones1d = jnp.ones((CHUNK,), jnp.float32)

    b1_2d = b1.reshape(1, D)
    g1_2d = g1.reshape(1, D)
    be1_2d = be1.reshape(1, D)
    b2_2d = b2.reshape(1, D)
    g2_2d = g2.reshape(1, D)
    be2_2d = be2.reshape(1, D)

    table1, xr = pl.pallas_call(
        _tc1_body,
        grid=grid,
        in_specs=[_row_spec(D), _full_spec((D, D)), _full_spec((D, D)),
                  _full_spec((1, D))],
        out_specs=[_row_spec(D), _row_spec(D)],
        out_shape=[jax.ShapeDtypeStruct((n, D), jnp.float32),
                   jax.ShapeDtypeStruct((n, D), jnp.float32)],
    )(x, W1_l, W1_r, b1_2d)

    part1, degp = _make_sc_segsum(ne_pad, True)(
        table1, src_p, dst_p, zeros2d, ones1d, zeros1d)
    d0 = degp[0, :n].reshape(n, 1)
    d1 = degp[1, :n].reshape(n, 1)

    table2, hr = pl.pallas_call(
        _tc2_body,
        grid=grid,
        in_specs=[_row_spec(D), _row_spec(D), _row_spec(1), _row_spec(1),
                  _row_spec(D), _full_spec((1, D)), _full_spec((1, D)),
                  _full_spec((D, D)), _full_spec((D, D)), _full_spec((1, D))],
        out_specs=[_row_spec(D), _row_spec(D)],
        out_shape=[jax.ShapeDtypeStruct((n, D), jnp.float32),
                   jax.ShapeDtypeStruct((n, D), jnp.float32)],
    )(part1[0], part1[1], d0, d1, xr, g1_2d, be1_2d, W2_l, W2_r, b2_2d)

    part2 = _make_sc_segsum(ne_pad, False)(
        table2, src_p, dst_p, zeros2d, ones1d, zeros1d)[0]

    out = pl.pallas_call(
        _tc3_body,
        grid=grid,
        in_specs=[_row_spec(D), _row_spec(D), _row_spec(1), _row_spec(1),
                  _row_spec(D), _full_spec((1, D)), _full_spec((1, D))],
        out_specs=_row_spec(D),
        out_shape=jax.ShapeDtypeStruct((n, D), jnp.float32),
    )(part2[0], part2[1], d0, d1, hr, g2_2d, be2_2d)

    return out
